# Initial kernel scaffold; baseline (speedup 1.0000x reference)
import jax
import jax.numpy as jnp
from jax.experimental import pallas as pl


def kernel(x, edge_index, adj_values, weight1, bias1, weight_out, bias_out):
    raise NotImplementedError("write your pallas kernel here")


# speedup vs baseline: 4.1577x; 4.1577x over previous
import functools

import jax
import jax.numpy as jnp
from jax import lax
from jax.experimental import pallas as pl
from jax.experimental.pallas import tpu as pltpu
from jax.experimental.pallas import tpu_sc as plsc

_N = 10000
_E = 320000
_NORM_SCALE = 1.0

_NC = 2
_NS = 16
_NW = _NC * _NS
_EW = _E // _NW
_C = 80
_NCHUNK = _EW // _C
_NP = 10240
_RT = _NP // _NS
_ZR = 128


def _make_spmm(F: int):
    mesh = plsc.VectorSubcoreMesh(core_axis_name="c", subcore_axis_name="s")

    @functools.partial(
        pl.kernel,
        mesh=mesh,
        out_type=jax.ShapeDtypeStruct((_NC, _NP, F), jnp.float32),
        scratch_types=[
            pltpu.VMEM((_C,), jnp.int32),
            pltpu.VMEM((_C,), jnp.int32),
            pltpu.VMEM((_C,), jnp.float32),
            pltpu.VMEM((_C, F), jnp.float32),
            pltpu.VMEM((_ZR, F), jnp.float32),
            pltpu.VMEM_SHARED((_NP, F), jnp.float32),
            pltpu.SemaphoreType.DMA,
        ],
    )
    def spmm(h_hbm, ei_hbm, adj_hbm, out_hbm,
             src_v, dst_v, vals_v, rows_v, zeros_v, acc_sh, sem):
        c = lax.axis_index("c")
        s = lax.axis_index("s")
        wid = c * _NS + s

        zvec = jnp.zeros((16,), jnp.float32)

        def zrow(r, carry):
            for j in range(F // 16):
                zeros_v[r, pl.ds(j * 16, 16)] = zvec
            return carry

        lax.fori_loop(0, _ZR, zrow, 0)
        for k in range(_RT // _ZR):
            pltpu.sync_copy(zeros_v, acc_sh.at[pl.ds(s * _RT + k * _ZR, _ZR)])
        plsc.subcore_barrier()

        ebase = wid * _EW

        def chunk(ci, carry):
            base = ebase + ci * _C
            pltpu.sync_copy(ei_hbm.at[pl.ds(_E + base, _C)], src_v)
            pltpu.sync_copy(ei_hbm.at[pl.ds(base, _C)], dst_v)
            pltpu.sync_copy(adj_hbm.at[pl.ds(base, _C)], vals_v)
            pltpu.async_copy(h_hbm.at[src_v], rows_v, sem).wait()

            def group(g, gcarry):
                v16 = vals_v[pl.ds(g * 16, 16)]
                for i in range(16):
                    r = g * 16 + i
                    vvec = v16[jnp.full((16,), i, jnp.int32)]
                    for j in range(F // 16):
                        seg = rows_v[r, pl.ds(j * 16, 16)]
                        rows_v[r, pl.ds(j * 16, 16)] = seg * vvec
                return gcarry

            lax.fori_loop(0, _C // 16, group, 0)
            pltpu.sync_copy(rows_v, acc_sh.at[dst_v], add=True)
            return carry

        lax.fori_loop(0, _NCHUNK, chunk, 0)
        plsc.subcore_barrier()

        r0 = s * _RT
        pltpu.sync_copy(acc_sh.at[pl.ds(r0, _RT)],
                        out_hbm.at[c, pl.ds(r0, _RT)])

    return spmm


_spmm128 = _make_spmm(128)


def _mm_body(x_ref, w_ref, o_ref):
    o_ref[...] = jnp.dot(x_ref[...], w_ref[...],
                         preferred_element_type=jnp.float32)


def _mid_body(p_ref, b_ref, o_ref):
    agg = p_ref[0, :_N] + p_ref[1, :_N] + b_ref[...]
    col_mean = jnp.mean(agg, axis=0, keepdims=True)
    xc = agg - col_mean
    rownorm_mean = jnp.sqrt(1e-06 + jnp.mean(jnp.sum(xc * xc, axis=1)))
    o_ref[...] = jnp.maximum(_NORM_SCALE * xc / rownorm_mean, 0.0)


def _fin_body(p_ref, w_ref, b_ref, o_ref):
    agg = p_ref[0, :_N] + p_ref[1, :_N]
    o_ref[...] = jnp.dot(agg, w_ref[...],
                         preferred_element_type=jnp.float32) + b_ref[...]


def kernel(x, edge_index, adj_values, weight1, bias1, weight_out, bias_out):
    h = pl.pallas_call(
        _mm_body,
        out_shape=jax.ShapeDtypeStruct((_N, 128), jnp.float32),
    )(x, weight1)
    ei_flat = edge_index.reshape(2 * _E)
    p1 = _spmm128(h, ei_flat, adj_values)
    h2 = pl.pallas_call(
        _mid_body,
        out_shape=jax.ShapeDtypeStruct((_N, 128), jnp.float32),
    )(p1, bias1)
    p2 = _spmm128(h2, ei_flat, adj_values)
    out = pl.pallas_call(
        _fin_body,
        out_shape=jax.ShapeDtypeStruct((_N, 64), jnp.float32),
    )(p2, weight_out, bias_out)
    return out
